# baseline (speedup 1.0000x reference)
import functools

import jax
import jax.numpy as jnp
from jax import lax
from jax.experimental import pallas as pl
from jax.experimental.pallas import tpu as pltpu
from jax.experimental.pallas import tpu_sc as plsc

_VOCAB = 100000
_EMBED = 64
_CTX = 20
_HIDDEN = 128
_NB = 10
_BLK = _VOCAB // _NB


def _sc_gather_sum(idx, embed):
    mesh = plsc.VectorSubcoreMesh(core_axis_name="c", subcore_axis_name="s")

    @functools.partial(
        pl.kernel,
        out_type=jax.ShapeDtypeStruct((1, _EMBED), jnp.float32),
        mesh=mesh,
        scratch_types=[
            pltpu.VMEM((_CTX + 16,), jnp.int32),
            pltpu.VMEM((_CTX, 8, _EMBED), jnp.float32),
            pltpu.VMEM((1, _EMBED), jnp.float32),
            pltpu.SemaphoreType.DMA,
        ],
    )
    def k(idx_hbm, embed_hbm, out_hbm, idx_v, rows_v, acc_v, sem):
        wid = lax.axis_index("s") * 2 + lax.axis_index("c")

        @pl.when(wid == 0)
        def _():
            pltpu.sync_copy(idx_hbm, idx_v.at[pl.ds(0, _CTX)])
            handles = []
            for r in range(_CTX):
                base = (idx_v[pl.ds(r, 16)][0] >> 3) * 8
                handles.append(pltpu.async_copy(
                    embed_hbm.at[pl.ds(base, 8), :], rows_v.at[r], sem))
            for h in handles:
                h.wait()
            for d in range(_EMBED // 16):
                acc = jnp.zeros((16,), jnp.float32)
                for r in range(_CTX):
                    sub = idx_v[pl.ds(r, 16)][0] & 7
                    acc = acc + rows_v[r, sub, pl.ds(d * 16, 16)]
                acc_v[0, pl.ds(d * 16, 16)] = acc
            pltpu.sync_copy(acc_v, out_hbm)

    return k(idx, embed)


def _tc_body(s_ref, w1_ref, b1_ref, w2_ref, b2_ref, out_ref):
    i = pl.program_id(0)
    h = lax.dot_general(s_ref[...], w1_ref[...], (((1,), (1,)), ((), ())),
                        preferred_element_type=jnp.float32)
    h = jnp.maximum(h + b1_ref[...], 0.0)
    logits = lax.dot_general(h, w2_ref[0], (((1,), (1,)), ((), ())),
                             preferred_element_type=jnp.float32)
    out_ref[pl.ds(i, 1), :] = logits + b2_ref[0]

    @pl.when(i == _NB - 1)
    def _():
        x = out_ref[...]
        m = jnp.max(x)
        lse = m + jnp.log(jnp.sum(jnp.exp(x - m)))
        out_ref[...] = x - lse


def _tc_dense(s, fc1_w, fc1_b, fc2_w, fc2_b):
    out = pl.pallas_call(
        _tc_body,
        grid=(_NB,),
        in_specs=[
            pl.BlockSpec((1, _EMBED), lambda i: (0, 0)),
            pl.BlockSpec((_HIDDEN, _EMBED), lambda i: (0, 0)),
            pl.BlockSpec((1, _HIDDEN), lambda i: (0, 0)),
            pl.BlockSpec((1, _BLK, _HIDDEN), lambda i: (i, 0, 0)),
            pl.BlockSpec((1, 1, _BLK), lambda i: (i, 0, 0)),
        ],
        out_specs=pl.BlockSpec((_NB, _BLK), lambda i: (0, 0)),
        out_shape=jax.ShapeDtypeStruct((_NB, _BLK), jnp.float32),
    )(
        s,
        fc1_w,
        fc1_b.reshape(1, _HIDDEN),
        fc2_w.reshape(_NB, _BLK, _HIDDEN),
        fc2_b.reshape(_NB, 1, _BLK),
    )
    return out.reshape(_VOCAB)


def kernel(inputs, embed, fc1_w, fc1_b, fc2_w, fc2_b):
    s = _sc_gather_sum(inputs.astype(jnp.int32), embed)
    return _tc_dense(s, fc1_w, fc1_b, fc2_w, fc2_b)


# speedup vs baseline: 1.0139x; 1.0139x over previous
import functools

import jax
import jax.numpy as jnp
from jax import lax
from jax.experimental import pallas as pl
from jax.experimental.pallas import tpu as pltpu
from jax.experimental.pallas import tpu_sc as plsc

_VOCAB = 100000
_EMBED = 64
_CTX = 20
_HIDDEN = 128
_NB = 10
_BLK = _VOCAB // _NB


def _sc_gather_sum(idx, embed):
    mesh = plsc.VectorSubcoreMesh(core_axis_name="c", subcore_axis_name="s")

    @functools.partial(
        pl.kernel,
        out_type=jax.ShapeDtypeStruct((1, _EMBED), jnp.float32),
        mesh=mesh,
        scratch_types=[
            pltpu.VMEM((_CTX + 16,), jnp.int32),
            pltpu.VMEM((_CTX, _EMBED), jnp.float32),
            pltpu.VMEM((1, _EMBED), jnp.float32),
            pltpu.SemaphoreType.DMA,
        ],
    )
    def k(idx_hbm, embed_hbm, out_hbm, idx_v, rows_v, acc_v, sem):
        wid = lax.axis_index("s") * 2 + lax.axis_index("c")

        @pl.when(wid == 0)
        def _():
            pltpu.sync_copy(idx_hbm, idx_v.at[pl.ds(0, _CTX)])
            handles = []
            for r in range(_CTX):
                v = idx_v[pl.ds(r, 16)][0]
                handles.append(pltpu.async_copy(
                    embed_hbm.at[pl.ds(v, 1), :], rows_v.at[pl.ds(r, 1), :], sem))
            for h in handles:
                h.wait()
            for d in range(_EMBED // 16):
                acc = jnp.zeros((16,), jnp.float32)
                for r in range(_CTX):
                    acc = acc + rows_v[r, pl.ds(d * 16, 16)]
                acc_v[0, pl.ds(d * 16, 16)] = acc
            pltpu.sync_copy(acc_v, out_hbm)

    return k(idx, embed)


def _tc_body(s_ref, w1_ref, b1_ref, w2_ref, b2_ref, out_ref):
    i = pl.program_id(0)
    h = lax.dot_general(s_ref[...], w1_ref[...], (((1,), (1,)), ((), ())),
                        preferred_element_type=jnp.float32)
    h = jnp.maximum(h + b1_ref[...], 0.0)
    logits = lax.dot_general(h, w2_ref[0], (((1,), (1,)), ((), ())),
                             preferred_element_type=jnp.float32)
    out_ref[pl.ds(i, 1), :] = logits + b2_ref[0]

    @pl.when(i == _NB - 1)
    def _():
        x = out_ref[...]
        m = jnp.max(x)
        lse = m + jnp.log(jnp.sum(jnp.exp(x - m)))
        out_ref[...] = x - lse


def _tc_dense(s, fc1_w, fc1_b, fc2_w, fc2_b):
    out = pl.pallas_call(
        _tc_body,
        grid=(_NB,),
        in_specs=[
            pl.BlockSpec((1, _EMBED), lambda i: (0, 0)),
            pl.BlockSpec((_HIDDEN, _EMBED), lambda i: (0, 0)),
            pl.BlockSpec((1, _HIDDEN), lambda i: (0, 0)),
            pl.BlockSpec((1, _BLK, _HIDDEN), lambda i: (i, 0, 0)),
            pl.BlockSpec((1, 1, _BLK), lambda i: (i, 0, 0)),
        ],
        out_specs=pl.BlockSpec((_NB, _BLK), lambda i: (0, 0)),
        out_shape=jax.ShapeDtypeStruct((_NB, _BLK), jnp.float32),
    )(
        s,
        fc1_w,
        fc1_b.reshape(1, _HIDDEN),
        fc2_w.reshape(_NB, _BLK, _HIDDEN),
        fc2_b.reshape(_NB, 1, _BLK),
    )
    return out.reshape(_VOCAB)


def kernel(inputs, embed, fc1_w, fc1_b, fc2_w, fc2_b):
    s = _sc_gather_sum(inputs.astype(jnp.int32), embed)
    return _tc_dense(s, fc1_w, fc1_b, fc2_w, fc2_b)


# speedup vs baseline: 1.7419x; 1.7181x over previous
import functools

import jax
import jax.numpy as jnp
from jax import lax
from jax.experimental import pallas as pl
from jax.experimental.pallas import tpu as pltpu
from jax.experimental.pallas import tpu_sc as plsc

_VOCAB = 100000
_EMBED = 64
_CTX = 20
_HIDDEN = 128
_NB = 10
_BLK = _VOCAB // _NB


def _sc_gather_sum(idx, embed_t):
    mesh = plsc.VectorSubcoreMesh(core_axis_name="c", subcore_axis_name="s")

    @functools.partial(
        pl.kernel,
        out_type=jax.ShapeDtypeStruct((1, _EMBED), jnp.float32),
        mesh=mesh,
        scratch_types=[
            pltpu.VMEM((_CTX + 16,), jnp.int32),
            pltpu.VMEM((2, _EMBED, 128), jnp.float32),
            pltpu.VMEM((1, _EMBED), jnp.float32),
            pltpu.VMEM((16, _EMBED), jnp.float32),
            pltpu.VMEM_SHARED((16, _EMBED), jnp.float32),
            pltpu.SemaphoreType.DMA,
        ],
        compiler_params=pltpu.CompilerParams(needs_layout_passes=False),
    )
    def k(idx_hbm, embed_hbm, out_hbm, idx_v, cols_v, acc_v, sum_v, shared,
          sem):
        core = lax.axis_index("c")
        sub = lax.axis_index("s")

        def fetch(slot, r):
            v = idx_v[pl.ds(r, 16)][0]
            off = pl.multiple_of((v >> 7) * 128, 128)
            h = pltpu.async_copy(
                embed_hbm.at[:, pl.ds(off, 128)], cols_v.at[slot], sem)
            return h, v & 127

        def select(slot, lane):
            parts = []
            for d in range(_EMBED // 16):
                dims = jax.lax.iota(jnp.int32, 16) + d * 16
                rr = jnp.full((16,), slot, jnp.int32)
                ll = jnp.full((16,), lane, jnp.int32)
                parts.append(plsc.load_gather(cols_v, [rr, dims, ll]))
            return parts

        @pl.when(core == 0)
        def _():
            pltpu.sync_copy(idx_hbm, idx_v.at[pl.ds(0, _CTX)])
            n2 = _CTX - 16

            h0, l0 = fetch(0, sub)

            @pl.when(sub < n2)
            def _():
                h1, l1 = fetch(1, sub + 16)
                h1.wait()
            h0.wait()
            acc = select(0, l0)

            @pl.when(sub < n2)
            def _():
                v1 = idx_v[pl.ds(sub + 16, 16)][0]
                extra = select(1, v1 & 127)
                for d in range(_EMBED // 16):
                    acc_v[0, pl.ds(d * 16, 16)] = acc[d] + extra[d]

            @pl.when(sub >= n2)
            def _():
                for d in range(_EMBED // 16):
                    acc_v[0, pl.ds(d * 16, 16)] = acc[d]

            pltpu.sync_copy(acc_v, shared.at[pl.ds(sub, 1), :])
            plsc.subcore_barrier()

            @pl.when(sub == 0)
            def _():
                pltpu.sync_copy(shared, sum_v)
                for d in range(_EMBED // 16):
                    tot = sum_v[0, pl.ds(d * 16, 16)]
                    for r in range(1, 16):
                        tot = tot + sum_v[r, pl.ds(d * 16, 16)]
                    acc_v[0, pl.ds(d * 16, 16)] = tot
                pltpu.sync_copy(acc_v, out_hbm)

    return k(idx, embed_t)


def _tc_body(s_ref, w1_ref, b1_ref, w2_ref, b2_ref, out_ref):
    i = pl.program_id(0)
    h = lax.dot_general(s_ref[...], w1_ref[...], (((1,), (1,)), ((), ())),
                        preferred_element_type=jnp.float32)
    h = jnp.maximum(h + b1_ref[...], 0.0)
    logits = lax.dot_general(h, w2_ref[0], (((1,), (1,)), ((), ())),
                             preferred_element_type=jnp.float32)
    out_ref[pl.ds(i, 1), :] = logits + b2_ref[0]

    @pl.when(i == _NB - 1)
    def _():
        x = out_ref[...]
        m = jnp.max(x)
        lse = m + jnp.log(jnp.sum(jnp.exp(x - m)))
        out_ref[...] = x - lse


def _tc_dense(s, fc1_w, fc1_b, fc2_w, fc2_b):
    out = pl.pallas_call(
        _tc_body,
        grid=(_NB,),
        in_specs=[
            pl.BlockSpec((1, _EMBED), lambda i: (0, 0)),
            pl.BlockSpec((_HIDDEN, _EMBED), lambda i: (0, 0)),
            pl.BlockSpec((1, _HIDDEN), lambda i: (0, 0)),
            pl.BlockSpec((1, _BLK, _HIDDEN), lambda i: (i, 0, 0)),
            pl.BlockSpec((1, 1, _BLK), lambda i: (i, 0, 0)),
        ],
        out_specs=pl.BlockSpec((_NB, _BLK), lambda i: (0, 0)),
        out_shape=jax.ShapeDtypeStruct((_NB, _BLK), jnp.float32),
    )(
        s,
        fc1_w,
        fc1_b.reshape(1, _HIDDEN),
        fc2_w.reshape(_NB, _BLK, _HIDDEN),
        fc2_b.reshape(_NB, 1, _BLK),
    )
    return out.reshape(_VOCAB)


def kernel(inputs, embed, fc1_w, fc1_b, fc2_w, fc2_b):
    s = _sc_gather_sum(inputs.astype(jnp.int32), embed.T)
    return _tc_dense(s, fc1_w, fc1_b, fc2_w, fc2_b)


# speedup vs baseline: 1.7607x; 1.0108x over previous
import functools

import jax
import jax.numpy as jnp
from jax import lax
from jax.experimental import pallas as pl
from jax.experimental.pallas import tpu as pltpu
from jax.experimental.pallas import tpu_sc as plsc

_VOCAB = 100000
_EMBED = 64
_CTX = 20
_HIDDEN = 128
_NB = 10
_BLK = _VOCAB // _NB


def _sc_gather_sum(idx, embed_t):
    mesh = plsc.VectorSubcoreMesh(core_axis_name="c", subcore_axis_name="s")

    @functools.partial(
        pl.kernel,
        out_type=jax.ShapeDtypeStruct((16, _EMBED), jnp.float32),
        mesh=mesh,
        scratch_types=[
            pltpu.VMEM((_CTX + 16,), jnp.int32),
            pltpu.VMEM((2, _EMBED, 128), jnp.float32),
            pltpu.VMEM((1, _EMBED), jnp.float32),
            pltpu.SemaphoreType.DMA,
        ],
        compiler_params=pltpu.CompilerParams(needs_layout_passes=False),
    )
    def k(idx_hbm, embed_hbm, out_hbm, idx_v, cols_v, acc_v, sem):
        core = lax.axis_index("c")
        sub = lax.axis_index("s")

        def idx_at(r):
            rr = jnp.full((16,), r, jnp.int32)
            return plsc.load_gather(idx_v, [rr])[0]

        def fetch(slot, r):
            v = idx_at(r)
            off = pl.multiple_of((v >> 7) * 128, 128)
            h = pltpu.async_copy(
                embed_hbm.at[:, pl.ds(off, 128)], cols_v.at[slot], sem)
            return h, v & 127

        def select(slot, lane):
            parts = []
            for d in range(_EMBED // 16):
                dims = jax.lax.iota(jnp.int32, 16) + d * 16
                rr = jnp.full((16,), slot, jnp.int32)
                ll = jnp.full((16,), lane, jnp.int32)
                parts.append(plsc.load_gather(cols_v, [rr, dims, ll]))
            return parts

        @pl.when(core == 0)
        def _():
            pltpu.sync_copy(idx_hbm, idx_v.at[pl.ds(0, _CTX)])
            n2 = _CTX - 16

            h0, l0 = fetch(0, sub)

            @pl.when(sub < n2)
            def _():
                h1, l1 = fetch(1, sub + 16)
                h1.wait()
            h0.wait()
            acc = select(0, l0)

            @pl.when(sub < n2)
            def _():
                v1 = idx_at(sub + 16)
                extra = select(1, v1 & 127)
                for d in range(_EMBED // 16):
                    acc_v[0, pl.ds(d * 16, 16)] = acc[d] + extra[d]

            @pl.when(sub >= n2)
            def _():
                for d in range(_EMBED // 16):
                    acc_v[0, pl.ds(d * 16, 16)] = acc[d]

            pltpu.sync_copy(acc_v, out_hbm.at[pl.ds(sub, 1), :])

    return k(idx, embed_t)


def _tc_body(s_ref, w1_ref, b1_ref, w2_ref, b2_ref, out_ref):
    i = pl.program_id(0)
    s = jnp.sum(s_ref[...], axis=0, keepdims=True)
    h = lax.dot_general(s, w1_ref[...], (((1,), (1,)), ((), ())),
                        preferred_element_type=jnp.float32)
    h = jnp.maximum(h + b1_ref[...], 0.0)
    logits = lax.dot_general(h, w2_ref[0], (((1,), (1,)), ((), ())),
                             preferred_element_type=jnp.float32)
    out_ref[pl.ds(i, 1), :] = logits + b2_ref[0]

    @pl.when(i == _NB - 1)
    def _():
        x = out_ref[...]
        m = jnp.max(x)
        lse = m + jnp.log(jnp.sum(jnp.exp(x - m)))
        out_ref[...] = x - lse


def _tc_dense(s, fc1_w, fc1_b, fc2_w, fc2_b):
    out = pl.pallas_call(
        _tc_body,
        grid=(_NB,),
        in_specs=[
            pl.BlockSpec((16, _EMBED), lambda i: (0, 0)),
            pl.BlockSpec((_HIDDEN, _EMBED), lambda i: (0, 0)),
            pl.BlockSpec((1, _HIDDEN), lambda i: (0, 0)),
            pl.BlockSpec((1, _BLK, _HIDDEN), lambda i: (i, 0, 0)),
            pl.BlockSpec((1, 1, _BLK), lambda i: (i, 0, 0)),
        ],
        out_specs=pl.BlockSpec((_NB, _BLK), lambda i: (0, 0)),
        out_shape=jax.ShapeDtypeStruct((_NB, _BLK), jnp.float32),
    )(
        s,
        fc1_w,
        fc1_b.reshape(1, _HIDDEN),
        fc2_w.reshape(_NB, _BLK, _HIDDEN),
        fc2_b.reshape(_NB, 1, _BLK),
    )
    return out.reshape(_VOCAB)


def kernel(inputs, embed, fc1_w, fc1_b, fc2_w, fc2_b):
    s = _sc_gather_sum(inputs.astype(jnp.int32), embed.T)
    return _tc_dense(s, fc1_w, fc1_b, fc2_w, fc2_b)
